# baseline (speedup 1.0000x reference)
import numpy as np
import jax
import jax.numpy as jnp
from jax import lax
from jax.experimental import pallas as pl
from jax.experimental.pallas import tpu as pltpu
from jax.experimental.pallas import tpu_sc as plsc

ORDER_K = 2
GRID_H = 32
GRID_W = 32
NCH = 64
NPTS = 65536
LANES = 16
NWORKERS = 32
PPW = NPTS // NWORKERS
CHUNK = 256
NCHUNKS = PPW // CHUNK
GROUPS = CHUNK // LANES

B0, B1 = -0.1, 0.1
INV_DX = float((GRID_H - ORDER_K) / (B1 - B0))
KSTRIDE = 128


def _make_span_table():
    interior = np.linspace(B0, B1, GRID_H - ORDER_K + 1)[1:-1]
    t = np.concatenate([np.full(ORDER_K + 1, B0), interior,
                        np.full(ORDER_K + 1, B1)]).astype(np.float32)
    tab = np.zeros((5, KSTRIDE), np.float32)
    for k in range(ORDER_K, GRID_H):
        tab[0, k] = t[k - 1]
        tab[1, k] = t[k]
        tab[2, k] = np.float32(1.0) / (t[k + 1] - t[k - 1])
        tab[3, k] = np.float32(1.0) / (t[k + 2] - t[k])
        tab[4, k] = np.float32(1.0) / (t[k + 1] - t[k])
    return tab.reshape(-1)


_SPAN_TABLE = _make_span_table()


def _weights(v, kv, kt_v):
    tm1 = plsc.load_gather(kt_v, [kv])
    tk0 = plsc.load_gather(kt_v, [kv + KSTRIDE])
    r0 = plsc.load_gather(kt_v, [kv + 2 * KSTRIDE])
    r1 = plsc.load_gather(kt_v, [kv + 3 * KSTRIDE])
    rb = plsc.load_gather(kt_v, [kv + 4 * KSTRIDE])
    a0 = (v - tm1) * r0
    a1 = (v - tk0) * r1
    b = (v - tk0) * rb
    w0 = (1.0 - b) * (1.0 - a0)
    w1 = (1.0 - b) * a0 + b * (1.0 - a1)
    w2 = b * a1
    return w0, w1, w2


def _span(v):
    u = (v - B0) * INV_DX
    m = u.astype(jnp.int32)
    return jnp.clip(m + ORDER_K, ORDER_K, GRID_H - 1)


def _sc_body(xy_hbm, tab_hbm, kt_hbm, out_hbm, xyr_v, xs_v, ys_v, tab_v, kt_v, obuf):
    wid = lax.axis_index("s") * 2 + lax.axis_index("c")
    base = wid * PPW
    pltpu.sync_copy(xy_hbm.at[pl.ds(base * 2, PPW * 2)], xyr_v)
    pltpu.sync_copy(tab_hbm, tab_v)
    pltpu.sync_copy(kt_hbm, kt_v)

    lane64 = lax.iota(jnp.int32, LANES) * NCH
    lane2 = lax.iota(jnp.int32, LANES) * 2

    def deint_body(q, carry):
        q16 = q * LANES
        xv = plsc.load_gather(xyr_v, [q16 * 2 + lane2])
        yv = plsc.load_gather(xyr_v, [q16 * 2 + 1 + lane2])
        xs_v[pl.ds(q16, LANES)] = xv
        ys_v[pl.ds(q16, LANES)] = yv
        return carry

    lax.fori_loop(0, PPW // LANES, deint_body, 0)

    def chunk_body(cb, carry):
        def group_body(g, carry2):
            p0 = cb * CHUNK + g * LANES
            x = xs_v[pl.ds(p0, LANES)]
            y = ys_v[pl.ds(p0, LANES)]
            kx = _span(x)
            ky = _span(y)
            wx0, wx1, wx2 = _weights(x, kx, kt_v)
            wy0, wy1, wy2 = _weights(y, ky, kt_v)

            ws = []
            idxs = []
            rb = kx * GRID_W + ky - (2 * GRID_W + 2)
            for i, wxi in enumerate((wx0, wx1, wx2)):
                for j, wyj in enumerate((wy0, wy1, wy2)):
                    ws.append(wxi * wyj)
                    idxs.append(rb + (i * GRID_W + j))

            sidx = g * (LANES * NCH) + lane64
            NP = GRID_H * GRID_W
            BLK = 16
            for blk in range(NCH // BLK):
                accs = []
                for cc in range(BLK):
                    ch = blk * BLK + cc
                    vals = [plsc.load_gather(tab_v, [ix + ch * NP])
                            for ix in idxs]
                    acc = ws[0] * vals[0]
                    for t in range(1, 9):
                        acc = acc + ws[t] * vals[t]
                    accs.append(acc)
                for cc in range(BLK):
                    plsc.store_scatter(obuf, [sidx + (blk * BLK + cc)],
                                       accs[cc])
            return carry2

        lax.fori_loop(0, GROUPS, group_body, 0)
        off = (base + cb * CHUNK) * NCH
        pltpu.sync_copy(obuf, out_hbm.at[pl.ds(off, CHUNK * NCH)])
        return carry

    lax.fori_loop(0, NCHUNKS, chunk_body, 0)


@jax.jit
def _spline_sc(xy_flat, tab, kt):
    mesh = plsc.VectorSubcoreMesh(core_axis_name="c", subcore_axis_name="s",
                                  num_cores=2, num_subcores=16)
    f = pl.kernel(
        _sc_body,
        out_type=jax.ShapeDtypeStruct((NPTS * NCH,), jnp.float32),
        mesh=mesh,
        compiler_params=pltpu.CompilerParams(needs_layout_passes=False),
        scratch_types=[
            pltpu.VMEM((PPW * 2,), jnp.float32),
            pltpu.VMEM((PPW,), jnp.float32),
            pltpu.VMEM((PPW,), jnp.float32),
            pltpu.VMEM((NCH * GRID_H * GRID_W,), jnp.float32),
            pltpu.VMEM((5 * KSTRIDE,), jnp.float32),
            pltpu.VMEM((CHUNK * NCH,), jnp.float32),
        ],
    )
    return f(xy_flat, tab, kt)


def kernel(xy, C):
    out = _spline_sc(xy.reshape(-1), C.reshape(-1), jnp.asarray(_SPAN_TABLE))
    return out.reshape(NPTS, NCH, 1)


# speedup vs baseline: 1.2003x; 1.2003x over previous
import numpy as np
import jax
import jax.numpy as jnp
from jax import lax
from jax.experimental import pallas as pl
from jax.experimental.pallas import tpu as pltpu
from jax.experimental.pallas import tpu_sc as plsc

ORDER_K = 2
GRID_H = 32
GRID_W = 32
NCH = 64
NPTS = 65536
LANES = 16
NWORKERS = 32
PPW = NPTS // NWORKERS
CHUNK = 256
NCHUNKS = PPW // CHUNK
GROUPS = CHUNK // LANES

B0, B1 = -0.1, 0.1
INV_DX = float((GRID_H - ORDER_K) / (B1 - B0))
KSTRIDE = 128


def _make_span_table():
    interior = np.linspace(B0, B1, GRID_H - ORDER_K + 1)[1:-1]
    t = np.concatenate([np.full(ORDER_K + 1, B0), interior,
                        np.full(ORDER_K + 1, B1)]).astype(np.float32)
    tab = np.zeros((5, KSTRIDE), np.float32)
    for k in range(ORDER_K, GRID_H):
        tab[0, k] = t[k - 1]
        tab[1, k] = t[k]
        tab[2, k] = np.float32(1.0) / (t[k + 1] - t[k - 1])
        tab[3, k] = np.float32(1.0) / (t[k + 2] - t[k])
        tab[4, k] = np.float32(1.0) / (t[k + 1] - t[k])
    return tab.reshape(-1)


_SPAN_TABLE = _make_span_table()


def _weights(v, kv, kt_v):
    tm1 = plsc.load_gather(kt_v, [kv])
    tk0 = plsc.load_gather(kt_v, [kv + KSTRIDE])
    r0 = plsc.load_gather(kt_v, [kv + 2 * KSTRIDE])
    r1 = plsc.load_gather(kt_v, [kv + 3 * KSTRIDE])
    rb = plsc.load_gather(kt_v, [kv + 4 * KSTRIDE])
    a0 = (v - tm1) * r0
    a1 = (v - tk0) * r1
    b = (v - tk0) * rb
    w0 = (1.0 - b) * (1.0 - a0)
    w1 = (1.0 - b) * a0 + b * (1.0 - a1)
    w2 = b * a1
    return w0, w1, w2


def _span(v):
    u = (v - B0) * INV_DX
    m = u.astype(jnp.int32)
    return jnp.clip(m + ORDER_K, ORDER_K, GRID_H - 1)


def _sc_body(xyt_hbm, tab_hbm, kt_hbm, out_hbm, xs_v, ys_v, tab_v, kt_v, obuf):
    wid = lax.axis_index("s") * 2 + lax.axis_index("c")
    base = wid * PPW
    pltpu.sync_copy(xyt_hbm.at[0, pl.ds(base, PPW)], xs_v)
    pltpu.sync_copy(xyt_hbm.at[1, pl.ds(base, PPW)], ys_v)
    pltpu.sync_copy(tab_hbm, tab_v)
    pltpu.sync_copy(kt_hbm, kt_v)

    lane64 = lax.iota(jnp.int32, LANES) * NCH

    def chunk_body(cb, carry):
        def group_body(g, carry2):
            p0 = cb * CHUNK + g * LANES
            x = xs_v[pl.ds(p0, LANES)]
            y = ys_v[pl.ds(p0, LANES)]
            kx = _span(x)
            ky = _span(y)
            wx0, wx1, wx2 = _weights(x, kx, kt_v)
            wy0, wy1, wy2 = _weights(y, ky, kt_v)

            ws = []
            idxs = []
            rb = kx * GRID_W + ky - (2 * GRID_W + 2)
            for i, wxi in enumerate((wx0, wx1, wx2)):
                for j, wyj in enumerate((wy0, wy1, wy2)):
                    ws.append(wxi * wyj)
                    idxs.append(rb + (i * GRID_W + j))

            sidx = g * (LANES * NCH) + lane64
            NP = GRID_H * GRID_W
            BLK = 16
            for blk in range(NCH // BLK):
                accs = []
                for cc in range(BLK):
                    ch = blk * BLK + cc
                    vals = [plsc.load_gather(tab_v, [ix + ch * NP])
                            for ix in idxs]
                    acc = ws[0] * vals[0]
                    for t in range(1, 9):
                        acc = acc + ws[t] * vals[t]
                    accs.append(acc)
                for cc in range(BLK):
                    plsc.store_scatter(obuf, [sidx + (blk * BLK + cc)],
                                       accs[cc])
            return carry2

        lax.fori_loop(0, GROUPS, group_body, 0)
        off = (base + cb * CHUNK) * NCH
        pltpu.sync_copy(obuf, out_hbm.at[pl.ds(off, CHUNK * NCH)])
        return carry

    lax.fori_loop(0, NCHUNKS, chunk_body, 0)


@jax.jit
def _spline_sc(xyt, tab, kt):
    mesh = plsc.VectorSubcoreMesh(core_axis_name="c", subcore_axis_name="s",
                                  num_cores=2, num_subcores=16)
    f = pl.kernel(
        _sc_body,
        out_type=jax.ShapeDtypeStruct((NPTS * NCH,), jnp.float32),
        mesh=mesh,
        compiler_params=pltpu.CompilerParams(needs_layout_passes=False),
        scratch_types=[
            pltpu.VMEM((PPW,), jnp.float32),
            pltpu.VMEM((PPW,), jnp.float32),
            pltpu.VMEM((NCH * GRID_H * GRID_W,), jnp.float32),
            pltpu.VMEM((5 * KSTRIDE,), jnp.float32),
            pltpu.VMEM((CHUNK * NCH,), jnp.float32),
        ],
    )
    return f(xyt, tab, kt)


def kernel(xy, C):
    out = _spline_sc(xy.T, C.reshape(-1), jnp.asarray(_SPAN_TABLE))
    return out.reshape(NPTS, NCH, 1)
